# baseline (speedup 1.0000x reference)
import jax
import jax.numpy as jnp
from jax import lax
from jax.experimental import pallas as pl
from jax.experimental.pallas import tpu as pltpu
from jax.experimental.pallas import tpu_sc as plsc

N = 16
P = 65536
F_DIM = 32
HIDDEN = 64
C_DIM = 32
RESO = 16
S = RESO ** 3
PADDING = 0.02
NPTS = N * P
PH = NPTS // 128
SH = S // 128

TP = 2048
TPH = TP // 128
NB = NPTS // TP
SC_C = 8192
CPH = SC_C // 128


def _mlp_body(pts_ref, feat_ref, fc0w_ref, fc0b_ref, fc1w_ref, fc1b_ref,
              scw_ref, ht_ref, idx2_ref, idxout_ref):
    hi = lax.Precision.HIGHEST
    feat = feat_ref[...]
    eye_f = jnp.eye(F_DIM, dtype=jnp.float32)
    featT = lax.dot_general(eye_f, feat, (((1,), (1,)), ((), ())),
                            precision=hi,
                            preferred_element_type=jnp.float32)
    relu_f = jnp.maximum(featT, 0.0)
    netT = lax.dot_general(fc0w_ref[...], relu_f, (((1,), (0,)), ((), ())),
                           precision=hi, preferred_element_type=jnp.float32)
    netT = netT + fc0b_ref[...].reshape(F_DIM, 1)
    relu_n = jnp.maximum(netT, 0.0)
    dxT = lax.dot_general(fc1w_ref[...], relu_n, (((1,), (0,)), ((), ())),
                          precision=hi, preferred_element_type=jnp.float32)
    dxT = dxT + fc1b_ref[...].reshape(HIDDEN, 1)
    xsT = lax.dot_general(scw_ref[...], featT, (((1,), (0,)), ((), ())),
                          precision=hi, preferred_element_type=jnp.float32)
    ht = xsT + dxT
    ht_ref[...] = ht.reshape(8, 8, TPH, 128).transpose(0, 2, 1, 3)

    pts = pts_ref[...]
    eye3 = jnp.eye(3, dtype=jnp.float32)
    ptsT = lax.dot_general(eye3, pts, (((1,), (1,)), ((), ())),
                           precision=hi,
                           preferred_element_type=jnp.float32)
    coord = (ptsT - 0.5) / (1.0 + PADDING) + 0.5
    coord = jnp.clip(coord, 0.0, 1.0 - 1e-6)
    i3 = jnp.clip(jnp.floor(coord * RESO).astype(jnp.int32), 0, RESO - 1)
    idx = i3[0:1, :] + RESO * i3[1:2, :] + (RESO * RESO) * i3[2:3, :]
    idx2_ref[...] = idx.reshape(TPH, 128)
    idxout_ref[...] = idx.reshape(1, 1, TP)


def _mlp_stage(pts2, feat2, fc0_w, fc0_b, fc1_w, fc1_b, sc_w):
    return pl.pallas_call(
        _mlp_body,
        grid=(NB,),
        in_specs=[
            pl.BlockSpec((TP, 3), lambda g: (g, 0)),
            pl.BlockSpec((TP, F_DIM), lambda g: (g, 0)),
            pl.BlockSpec((F_DIM, F_DIM), lambda g: (0, 0)),
            pl.BlockSpec((1, F_DIM), lambda g: (0, 0)),
            pl.BlockSpec((HIDDEN, F_DIM), lambda g: (0, 0)),
            pl.BlockSpec((1, HIDDEN), lambda g: (0, 0)),
            pl.BlockSpec((HIDDEN, F_DIM), lambda g: (0, 0)),
        ],
        out_specs=[
            pl.BlockSpec((8, TPH, 8, 128), lambda g: (0, g, 0, 0)),
            pl.BlockSpec((TPH, 128), lambda g: (g, 0)),
            pl.BlockSpec((1, 1, TP),
                         lambda g: (g // (P // TP), 0, g % (P // TP))),
        ],
        out_shape=[
            jax.ShapeDtypeStruct((8, PH, 8, 128), jnp.float32),
            jax.ShapeDtypeStruct((PH, 128), jnp.int32),
            jax.ShapeDtypeStruct((N, 1, P), jnp.int32),
        ],
        compiler_params=pltpu.CompilerParams(
            dimension_semantics=("arbitrary",)),
    )(pts2, feat2, fc0_w, fc0_b.reshape(1, F_DIM), fc1_w,
      fc1_b.reshape(1, HIDDEN), sc_w)


BANKS = 8
BS = BANKS * S


def _scatter_max_body(ht_hbm, idx_hbm, out_hbm, acc0, acc1, idxbuf, vbuf,
                      outbuf0, outbuf1):
    cid = lax.axis_index("c")
    sid = lax.axis_index("s")
    wid = sid * 2 + cid
    dh = wid // 4
    dl0 = (wid % 4) * 2
    neg = jnp.full((16,), -jnp.inf, dtype=jnp.float32)
    lane = lax.iota(jnp.int32, 16)
    bankoff = jnp.minimum(lane, 15 - lane) * S

    def batch_body(n, _):
        def init_body(j, _):
            for k in range(8):
                sl = pl.ds(j * 128 + k * 16, 16)
                acc0[sl] = neg
                acc1[sl] = neg
            return 0
        lax.fori_loop(0, BS // 128, init_body, 0)

        def chunk_body(cc, _):
            phbase = n * (P // 128) + cc * CPH
            pltpu.sync_copy(idx_hbm.at[pl.ds(phbase, CPH), :], idxbuf)
            pltpu.sync_copy(
                ht_hbm.at[dh, pl.ds(phbase, CPH), pl.ds(dl0, 2), :], vbuf)

            def ph_body(ph, _):
                for j in range(8):
                    sl = pl.ds(j * 16, 16)
                    idxv = idxbuf[ph, sl]
                    v0 = vbuf[ph, 0, sl]
                    v1 = vbuf[ph, 1, sl]
                    eq = jnp.flip(idxv) == idxv
                    v0 = jnp.where(eq, jnp.maximum(v0, jnp.flip(v0)), v0)
                    v1 = jnp.where(eq, jnp.maximum(v1, jnp.flip(v1)), v1)
                    addr = idxv + bankoff
                    c0 = plsc.load_gather(acc0, [addr])
                    c1 = plsc.load_gather(acc1, [addr])
                    plsc.store_scatter(acc0, [addr], jnp.maximum(c0, v0))
                    plsc.store_scatter(acc1, [addr], jnp.maximum(c1, v1))
                return 0

            lax.fori_loop(0, CPH, ph_body, 0)
            return 0

        lax.fori_loop(0, P // SC_C, chunk_body, 0)

        def red_body(sh, _):
            for j in range(8):
                pos = sh * 128 + j * 16
                m0 = acc0[pl.ds(pos, 16)]
                m1 = acc1[pl.ds(pos, 16)]
                for b in range(1, BANKS):
                    m0 = jnp.maximum(m0, acc0[pl.ds(b * S + pos, 16)])
                    m1 = jnp.maximum(m1, acc1[pl.ds(b * S + pos, 16)])
                outbuf0[sh, pl.ds(j * 16, 16)] = m0
                outbuf1[sh, pl.ds(j * 16, 16)] = m1
            return 0
        lax.fori_loop(0, SH, red_body, 0)
        pltpu.sync_copy(outbuf0, out_hbm.at[n, dh, :, dl0, :])
        pltpu.sync_copy(outbuf1, out_hbm.at[n, dh, :, dl0 + 1, :])
        return 0

    lax.fori_loop(0, N, batch_body, 0)


def _scatter_max_stage(ht4, idx2):
    mesh = plsc.VectorSubcoreMesh(core_axis_name="c", subcore_axis_name="s")
    f = pl.kernel(
        _scatter_max_body,
        out_type=jax.ShapeDtypeStruct((N, 8, SH, 8, 128), jnp.float32),
        mesh=mesh,
        scratch_types=[
            pltpu.VMEM((BS,), jnp.float32),
            pltpu.VMEM((BS,), jnp.float32),
            pltpu.VMEM((CPH, 128), jnp.int32),
            pltpu.VMEM((CPH, 2, 128), jnp.float32),
            pltpu.VMEM((SH, 128), jnp.float32),
            pltpu.VMEM((SH, 128), jnp.float32),
        ],
        compiler_params=pltpu.CompilerParams(needs_layout_passes=False,
                                             use_tc_tiling_on_sc=True),
    )
    return f(ht4, idx2)


def _conv_body(pooled_ref, convw_ref, convb_ref, out_ref):
    blk = pooled_ref[...]
    p = blk.reshape(8, SH, 8, 128).transpose(0, 2, 1, 3).reshape(HIDDEN, S)
    p = jnp.where(p == -jnp.inf, 0.0, p)
    g = lax.dot_general(convw_ref[...], p, (((1,), (0,)), ((), ())),
                        precision=lax.Precision.HIGHEST,
                        preferred_element_type=jnp.float32)
    g = g + convb_ref[...].reshape(C_DIM, 1)
    out_ref[...] = g.reshape(1, C_DIM, S)


def _conv_stage(pooled5, conv_w, conv_b):
    return pl.pallas_call(
        _conv_body,
        grid=(N,),
        in_specs=[
            pl.BlockSpec((1, 8, SH, 8, 128), lambda n: (n, 0, 0, 0, 0)),
            pl.BlockSpec((C_DIM, HIDDEN), lambda n: (0, 0)),
            pl.BlockSpec((1, C_DIM), lambda n: (0, 0)),
        ],
        out_specs=pl.BlockSpec((1, C_DIM, S), lambda n: (n, 0, 0)),
        out_shape=jax.ShapeDtypeStruct((N, C_DIM, S), jnp.float32),
        compiler_params=pltpu.CompilerParams(
            dimension_semantics=("arbitrary",)),
    )(pooled5, conv_w, conv_b.reshape(1, C_DIM))


def kernel(normalized_points, feature, fc0_w, fc0_b, fc1_w, fc1_b, sc_w,
           conv_w, conv_b):
    pts2 = normalized_points.reshape(NPTS, 3)
    feat2 = feature.reshape(NPTS, F_DIM)
    ht4, idx2, index_out = _mlp_stage(pts2, feat2, fc0_w, fc0_b, fc1_w,
                                      fc1_b, sc_w)
    pooled5 = _scatter_max_stage(ht4, idx2)
    grid = _conv_stage(pooled5, conv_w, conv_b)
    return (index_out, grid.reshape(N, C_DIM, RESO, RESO, RESO))


# speedup vs baseline: 3.0989x; 3.0989x over previous
import jax
import jax.numpy as jnp
from jax import lax
from jax.experimental import pallas as pl
from jax.experimental.pallas import tpu as pltpu
from jax.experimental.pallas import tpu_sc as plsc

N = 16
P = 65536
F_DIM = 32
HIDDEN = 64
C_DIM = 32
RESO = 16
S = RESO ** 3
PADDING = 0.02
NPTS = N * P
SH = S // 128

NBG = P // 128
GC = 64


def _mlp_body(pts_ref, feat_ref, fc0w_ref, fc0b_ref, fc1w_ref, fc1b_ref,
              scw_ref, ht_ref, idx2_ref, idxout_ref):
    hi = lax.Precision.HIGHEST
    featT = feat_ref[...].transpose(1, 0, 2).reshape(F_DIM, N * 128)
    relu_f = jnp.maximum(featT, 0.0)
    netT = lax.dot_general(fc0w_ref[...], relu_f, (((1,), (0,)), ((), ())),
                           precision=hi, preferred_element_type=jnp.float32)
    netT = netT + fc0b_ref[...].reshape(F_DIM, 1)
    relu_n = jnp.maximum(netT, 0.0)
    dxT = lax.dot_general(fc1w_ref[...], relu_n, (((1,), (0,)), ((), ())),
                          precision=hi, preferred_element_type=jnp.float32)
    dxT = dxT + fc1b_ref[...].reshape(HIDDEN, 1)
    xsT = lax.dot_general(scw_ref[...], featT, (((1,), (0,)), ((), ())),
                          precision=hi, preferred_element_type=jnp.float32)
    ht = xsT + dxT
    ht_ref[...] = (ht.reshape(8, 8, N, 128).transpose(0, 2, 1, 3)
                   .reshape(8, 1, N, 8, 128))

    pts = pts_ref[...]
    coord = (pts - 0.5) / (1.0 + PADDING) + 0.5
    coord = jnp.clip(coord, 0.0, 1.0 - 1e-6)
    i3 = jnp.clip(jnp.floor(coord * RESO).astype(jnp.int32), 0, RESO - 1)
    idx = (i3[0:1] + RESO * i3[1:2] + (RESO * RESO) * i3[2:3])
    idx2_ref[...] = idx.reshape(1, N, 128)
    idxout_ref[...] = idx.reshape(N, 1, 128)


def _mlp_stage(pts3, feat3, fc0_w, fc0_b, fc1_w, fc1_b, sc_w):
    return pl.pallas_call(
        _mlp_body,
        grid=(NBG,),
        in_specs=[
            pl.BlockSpec((3, N, 128), lambda g: (0, 0, g)),
            pl.BlockSpec((N, F_DIM, 128), lambda g: (0, 0, g)),
            pl.BlockSpec((F_DIM, F_DIM), lambda g: (0, 0)),
            pl.BlockSpec((1, F_DIM), lambda g: (0, 0)),
            pl.BlockSpec((HIDDEN, F_DIM), lambda g: (0, 0)),
            pl.BlockSpec((1, HIDDEN), lambda g: (0, 0)),
            pl.BlockSpec((HIDDEN, F_DIM), lambda g: (0, 0)),
        ],
        out_specs=[
            pl.BlockSpec((8, 1, N, 8, 128), lambda g: (0, g, 0, 0, 0)),
            pl.BlockSpec((1, N, 128), lambda g: (g, 0, 0)),
            pl.BlockSpec((N, 1, 128), lambda g: (0, 0, g)),
        ],
        out_shape=[
            jax.ShapeDtypeStruct((8, NBG, N, 8, 128), jnp.float32),
            jax.ShapeDtypeStruct((NBG, N, 128), jnp.int32),
            jax.ShapeDtypeStruct((N, 1, P), jnp.int32),
        ],
        compiler_params=pltpu.CompilerParams(
            dimension_semantics=("arbitrary",)),
    )(pts3, feat3, fc0_w, fc0_b.reshape(1, F_DIM), fc1_w,
      fc1_b.reshape(1, HIDDEN), sc_w)


BANKS = 8
BS = BANKS * S


def _scatter_max_body(ht_hbm, idx_hbm, out_hbm, acc0, acc1, idxbuf, vbuf,
                      outbuf0, outbuf1):
    cid = lax.axis_index("c")
    sid = lax.axis_index("s")
    wid = sid * 2 + cid
    dh = wid // 4
    dl0 = (wid % 4) * 2
    neg = jnp.full((16,), -jnp.inf, dtype=jnp.float32)
    lane = lax.iota(jnp.int32, 16)
    bankoff = jnp.minimum(lane, 15 - lane) * S

    def batch_body(n, _):
        def init_body(j, _):
            for k in range(8):
                sl = pl.ds(j * 128 + k * 16, 16)
                acc0[sl] = neg
                acc1[sl] = neg
            return 0
        lax.fori_loop(0, BS // 128, init_body, 0)

        def chunk_body(cc, _):
            g0 = cc * GC
            pltpu.sync_copy(idx_hbm.at[pl.ds(g0, GC), n, :], idxbuf)
            pltpu.sync_copy(
                ht_hbm.at[dh, pl.ds(g0, GC), n, pl.ds(dl0, 2), :], vbuf)

            def ph_body(ph, _):
                for j in range(8):
                    sl = pl.ds(j * 16, 16)
                    idxv = idxbuf[ph, sl]
                    v0 = vbuf[ph, 0, sl]
                    v1 = vbuf[ph, 1, sl]
                    eq = jnp.flip(idxv) == idxv
                    v0 = jnp.where(eq, jnp.maximum(v0, jnp.flip(v0)), v0)
                    v1 = jnp.where(eq, jnp.maximum(v1, jnp.flip(v1)), v1)
                    addr = idxv + bankoff
                    c0 = plsc.load_gather(acc0, [addr])
                    c1 = plsc.load_gather(acc1, [addr])
                    plsc.store_scatter(acc0, [addr], jnp.maximum(c0, v0))
                    plsc.store_scatter(acc1, [addr], jnp.maximum(c1, v1))
                return 0

            lax.fori_loop(0, GC, ph_body, 0)
            return 0

        lax.fori_loop(0, NBG // GC, chunk_body, 0)

        def red_body(sh, _):
            for j in range(8):
                pos = sh * 128 + j * 16
                m0 = acc0[pl.ds(pos, 16)]
                m1 = acc1[pl.ds(pos, 16)]
                for b in range(1, BANKS):
                    m0 = jnp.maximum(m0, acc0[pl.ds(b * S + pos, 16)])
                    m1 = jnp.maximum(m1, acc1[pl.ds(b * S + pos, 16)])
                outbuf0[sh, pl.ds(j * 16, 16)] = m0
                outbuf1[sh, pl.ds(j * 16, 16)] = m1
            return 0
        lax.fori_loop(0, SH, red_body, 0)
        pltpu.sync_copy(outbuf0, out_hbm.at[n, dh, :, dl0, :])
        pltpu.sync_copy(outbuf1, out_hbm.at[n, dh, :, dl0 + 1, :])
        return 0

    lax.fori_loop(0, N, batch_body, 0)


def _scatter_max_stage(ht5, idx3):
    mesh = plsc.VectorSubcoreMesh(core_axis_name="c", subcore_axis_name="s")
    f = pl.kernel(
        _scatter_max_body,
        out_type=jax.ShapeDtypeStruct((N, 8, SH, 8, 128), jnp.float32),
        mesh=mesh,
        scratch_types=[
            pltpu.VMEM((BS,), jnp.float32),
            pltpu.VMEM((BS,), jnp.float32),
            pltpu.VMEM((GC, 128), jnp.int32),
            pltpu.VMEM((GC, 2, 128), jnp.float32),
            pltpu.VMEM((SH, 128), jnp.float32),
            pltpu.VMEM((SH, 128), jnp.float32),
        ],
        compiler_params=pltpu.CompilerParams(needs_layout_passes=False),
    )
    return f(ht5, idx3)


def _conv_body(pooled_ref, convw_ref, convb_ref, out_ref):
    blk = pooled_ref[...]
    p = blk.reshape(8, SH, 8, 128).transpose(0, 2, 1, 3).reshape(HIDDEN, S)
    p = jnp.where(p == -jnp.inf, 0.0, p)
    g = lax.dot_general(convw_ref[...], p, (((1,), (0,)), ((), ())),
                        precision=lax.Precision.HIGHEST,
                        preferred_element_type=jnp.float32)
    g = g + convb_ref[...].reshape(C_DIM, 1)
    out_ref[...] = g.reshape(1, C_DIM, S)


def _conv_stage(pooled5, conv_w, conv_b):
    return pl.pallas_call(
        _conv_body,
        grid=(N,),
        in_specs=[
            pl.BlockSpec((1, 8, SH, 8, 128), lambda n: (n, 0, 0, 0, 0)),
            pl.BlockSpec((C_DIM, HIDDEN), lambda n: (0, 0)),
            pl.BlockSpec((1, C_DIM), lambda n: (0, 0)),
        ],
        out_specs=pl.BlockSpec((1, C_DIM, S), lambda n: (n, 0, 0)),
        out_shape=jax.ShapeDtypeStruct((N, C_DIM, S), jnp.float32),
        compiler_params=pltpu.CompilerParams(
            dimension_semantics=("arbitrary",)),
    )(pooled5, conv_w, conv_b.reshape(1, C_DIM))


def kernel(normalized_points, feature, fc0_w, fc0_b, fc1_w, fc1_b, sc_w,
           conv_w, conv_b):
    pts3 = jnp.transpose(normalized_points, (2, 0, 1))
    feat3 = jnp.transpose(feature, (0, 2, 1))
    ht5, idx3, index_out = _mlp_stage(pts3, feat3, fc0_w, fc0_b, fc1_w,
                                      fc1_b, sc_w)
    pooled5 = _scatter_max_stage(ht5, idx3)
    grid = _conv_stage(pooled5, conv_w, conv_b)
    return (index_out, grid.reshape(N, C_DIM, RESO, RESO, RESO))


# speedup vs baseline: 3.6530x; 1.1788x over previous
import jax
import jax.numpy as jnp
from jax import lax
from jax.experimental import pallas as pl
from jax.experimental.pallas import tpu as pltpu
from jax.experimental.pallas import tpu_sc as plsc

N = 16
P = 65536
F_DIM = 32
HIDDEN = 64
C_DIM = 32
RESO = 16
S = RESO ** 3
PADDING = 0.02
NPTS = N * P
SH = S // 128

NBG = P // 128
GC = 64


def _mlp_body(pts_ref, feat_ref, fc0w_ref, fc0b_ref, fc1w_ref, fc1b_ref,
              scw_ref, ht_ref, idx2_ref, idxout_ref):
    hi = lax.Precision.HIGHEST
    featT = feat_ref[...].transpose(1, 0, 2).reshape(F_DIM, N * 128)
    relu_f = jnp.maximum(featT, 0.0)
    netT = lax.dot_general(fc0w_ref[...], relu_f, (((1,), (0,)), ((), ())),
                           precision=hi, preferred_element_type=jnp.float32)
    netT = netT + fc0b_ref[...].reshape(F_DIM, 1)
    relu_n = jnp.maximum(netT, 0.0)
    dxT = lax.dot_general(fc1w_ref[...], relu_n, (((1,), (0,)), ((), ())),
                          precision=hi, preferred_element_type=jnp.float32)
    dxT = dxT + fc1b_ref[...].reshape(HIDDEN, 1)
    xsT = lax.dot_general(scw_ref[...], featT, (((1,), (0,)), ((), ())),
                          precision=hi, preferred_element_type=jnp.float32)
    ht = xsT + dxT
    ht_ref[...] = (ht.reshape(8, 8, N, 128).transpose(0, 2, 1, 3)
                   .reshape(8, 1, N, 8, 128))

    pts = pts_ref[...]
    coord = (pts - 0.5) / (1.0 + PADDING) + 0.5
    coord = jnp.clip(coord, 0.0, 1.0 - 1e-6)
    i3 = jnp.clip(jnp.floor(coord * RESO).astype(jnp.int32), 0, RESO - 1)
    idx = (i3[0:1] + RESO * i3[1:2] + (RESO * RESO) * i3[2:3])
    idx2_ref[...] = idx.reshape(1, N, 128)
    idxout_ref[...] = idx.reshape(N, 1, 128)


def _mlp_stage(pts3, feat3, fc0_w, fc0_b, fc1_w, fc1_b, sc_w):
    return pl.pallas_call(
        _mlp_body,
        grid=(NBG,),
        in_specs=[
            pl.BlockSpec((3, N, 128), lambda g: (0, 0, g)),
            pl.BlockSpec((N, F_DIM, 128), lambda g: (0, 0, g)),
            pl.BlockSpec((F_DIM, F_DIM), lambda g: (0, 0)),
            pl.BlockSpec((1, F_DIM), lambda g: (0, 0)),
            pl.BlockSpec((HIDDEN, F_DIM), lambda g: (0, 0)),
            pl.BlockSpec((1, HIDDEN), lambda g: (0, 0)),
            pl.BlockSpec((HIDDEN, F_DIM), lambda g: (0, 0)),
        ],
        out_specs=[
            pl.BlockSpec((8, 1, N, 8, 128), lambda g: (0, g, 0, 0, 0)),
            pl.BlockSpec((1, N, 128), lambda g: (g, 0, 0)),
            pl.BlockSpec((N, 1, 128), lambda g: (0, 0, g)),
        ],
        out_shape=[
            jax.ShapeDtypeStruct((8, NBG, N, 8, 128), jnp.float32),
            jax.ShapeDtypeStruct((NBG, N, 128), jnp.int32),
            jax.ShapeDtypeStruct((N, 1, P), jnp.int32),
        ],
        compiler_params=pltpu.CompilerParams(
            dimension_semantics=("arbitrary",)),
    )(pts3, feat3, fc0_w, fc0_b.reshape(1, F_DIM), fc1_w,
      fc1_b.reshape(1, HIDDEN), sc_w)


BANKS = 8
BS = BANKS * S


CH = NBG // GC
NCH = N * CH


def _scatter_max_body(ht_hbm, idx_hbm, out_hbm, acc0, acc1, idxbuf, vbuf,
                      outbuf0, outbuf1, sem):
    cid = lax.axis_index("c")
    sid = lax.axis_index("s")
    wid = sid * 2 + cid
    dh = wid // 4
    dl0 = (wid % 4) * 2
    neg = jnp.full((16,), -jnp.inf, dtype=jnp.float32)
    lane = lax.iota(jnp.int32, 16)
    bankoff = jnp.minimum(lane, 15 - lane) * S

    def _slices(t):
        par = t % 2
        tn = t // CH
        g0 = (t % CH) * GC
        return (par, idx_hbm.at[pl.ds(g0, GC), tn, :],
                ht_hbm.at[dh, pl.ds(g0, GC), tn, pl.ds(dl0, 2), :])

    def issue(t):
        par, isrc, vsrc = _slices(t)
        pltpu.async_copy(isrc, idxbuf.at[par], sem.at[par])
        pltpu.async_copy(vsrc, vbuf.at[par], sem.at[par])

    def wait(t):
        par, isrc, vsrc = _slices(t)
        pltpu.make_async_copy(isrc, idxbuf.at[par], sem.at[par]).wait()
        pltpu.make_async_copy(vsrc, vbuf.at[par], sem.at[par]).wait()
        return par

    issue(0)

    def batch_body(n, _):
        def init_body(j, _):
            for k in range(8):
                sl = pl.ds(j * 128 + k * 16, 16)
                acc0[sl] = neg
                acc1[sl] = neg
            return 0
        lax.fori_loop(0, BS // 128, init_body, 0)

        def chunk_body(cc, _):
            t = n * CH + cc

            @pl.when(t + 1 < NCH)
            def _():
                issue(t + 1)

            par = wait(t)

            def ph_body(ph, _):
                for j in range(8):
                    sl = pl.ds(j * 16, 16)
                    idxv = idxbuf[par, ph, sl]
                    v0 = vbuf[par, ph, 0, sl]
                    v1 = vbuf[par, ph, 1, sl]
                    eq = jnp.flip(idxv) == idxv
                    v0 = jnp.where(eq, jnp.maximum(v0, jnp.flip(v0)), v0)
                    v1 = jnp.where(eq, jnp.maximum(v1, jnp.flip(v1)), v1)
                    addr = idxv + bankoff
                    c0 = plsc.load_gather(acc0, [addr])
                    c1 = plsc.load_gather(acc1, [addr])
                    plsc.store_scatter(acc0, [addr], jnp.maximum(c0, v0))
                    plsc.store_scatter(acc1, [addr], jnp.maximum(c1, v1))
                return 0

            lax.fori_loop(0, GC, ph_body, 0)
            return 0

        lax.fori_loop(0, CH, chunk_body, 0)

        def red_body(sh, _):
            for j in range(8):
                pos = sh * 128 + j * 16
                m0 = acc0[pl.ds(pos, 16)]
                m1 = acc1[pl.ds(pos, 16)]
                for b in range(1, BANKS):
                    m0 = jnp.maximum(m0, acc0[pl.ds(b * S + pos, 16)])
                    m1 = jnp.maximum(m1, acc1[pl.ds(b * S + pos, 16)])
                outbuf0[sh, pl.ds(j * 16, 16)] = m0
                outbuf1[sh, pl.ds(j * 16, 16)] = m1
            return 0
        lax.fori_loop(0, SH, red_body, 0)
        pltpu.sync_copy(outbuf0, out_hbm.at[n, dh, :, dl0, :])
        pltpu.sync_copy(outbuf1, out_hbm.at[n, dh, :, dl0 + 1, :])
        return 0

    lax.fori_loop(0, N, batch_body, 0)


def _scatter_max_stage(ht5, idx3):
    mesh = plsc.VectorSubcoreMesh(core_axis_name="c", subcore_axis_name="s")
    f = pl.kernel(
        _scatter_max_body,
        out_type=jax.ShapeDtypeStruct((N, 8, SH, 8, 128), jnp.float32),
        mesh=mesh,
        scratch_types=[
            pltpu.VMEM((BS,), jnp.float32),
            pltpu.VMEM((BS,), jnp.float32),
            pltpu.VMEM((2, GC, 128), jnp.int32),
            pltpu.VMEM((2, GC, 2, 128), jnp.float32),
            pltpu.VMEM((SH, 128), jnp.float32),
            pltpu.VMEM((SH, 128), jnp.float32),
            pltpu.SemaphoreType.DMA((2,)),
        ],
        compiler_params=pltpu.CompilerParams(needs_layout_passes=False),
    )
    return f(ht5, idx3)


def _conv_body(pooled_ref, convw_ref, convb_ref, out_ref):
    blk = pooled_ref[...]
    p = blk.reshape(8, SH, 8, 128).transpose(0, 2, 1, 3).reshape(HIDDEN, S)
    p = jnp.where(p == -jnp.inf, 0.0, p)
    g = lax.dot_general(convw_ref[...], p, (((1,), (0,)), ((), ())),
                        precision=lax.Precision.HIGHEST,
                        preferred_element_type=jnp.float32)
    g = g + convb_ref[...].reshape(C_DIM, 1)
    out_ref[...] = g.reshape(1, C_DIM, S)


def _conv_stage(pooled5, conv_w, conv_b):
    return pl.pallas_call(
        _conv_body,
        grid=(N,),
        in_specs=[
            pl.BlockSpec((1, 8, SH, 8, 128), lambda n: (n, 0, 0, 0, 0)),
            pl.BlockSpec((C_DIM, HIDDEN), lambda n: (0, 0)),
            pl.BlockSpec((1, C_DIM), lambda n: (0, 0)),
        ],
        out_specs=pl.BlockSpec((1, C_DIM, S), lambda n: (n, 0, 0)),
        out_shape=jax.ShapeDtypeStruct((N, C_DIM, S), jnp.float32),
        compiler_params=pltpu.CompilerParams(
            dimension_semantics=("arbitrary",)),
    )(pooled5, conv_w, conv_b.reshape(1, C_DIM))


def kernel(normalized_points, feature, fc0_w, fc0_b, fc1_w, fc1_b, sc_w,
           conv_w, conv_b):
    pts3 = jnp.transpose(normalized_points, (2, 0, 1))
    feat3 = jnp.transpose(feature, (0, 2, 1))
    ht5, idx3, index_out = _mlp_stage(pts3, feat3, fc0_w, fc0_b, fc1_w,
                                      fc1_b, sc_w)
    pooled5 = _scatter_max_stage(ht5, idx3)
    grid = _conv_stage(pooled5, conv_w, conv_b)
    return (index_out, grid.reshape(N, C_DIM, RESO, RESO, RESO))


# speedup vs baseline: 4.2989x; 1.1768x over previous
import jax
import jax.numpy as jnp
from jax import lax
from jax.experimental import pallas as pl
from jax.experimental.pallas import tpu as pltpu
from jax.experimental.pallas import tpu_sc as plsc

N = 16
P = 65536
F_DIM = 32
HIDDEN = 64
C_DIM = 32
RESO = 16
S = RESO ** 3
PADDING = 0.02
NPTS = N * P
SH = S // 128

NBG = P // 128
HALF = NBG // 2
GC = 64


def _mlp_body(pts_ref, feat_ref, fc0w_ref, fc0b_ref, fc1w_ref, fc1b_ref,
              scw_ref, ht_ref, idx2_ref, idxout_ref):
    hi = lax.Precision.HIGHEST
    featT = feat_ref[...].transpose(1, 0, 2).reshape(F_DIM, N * 128)
    relu_f = jnp.maximum(featT, 0.0)
    netT = lax.dot_general(fc0w_ref[...], relu_f, (((1,), (0,)), ((), ())),
                           precision=hi, preferred_element_type=jnp.float32)
    netT = netT + fc0b_ref[...].reshape(F_DIM, 1)
    relu_n = jnp.maximum(netT, 0.0)
    dxT = lax.dot_general(fc1w_ref[...], relu_n, (((1,), (0,)), ((), ())),
                          precision=hi, preferred_element_type=jnp.float32)
    dxT = dxT + fc1b_ref[...].reshape(HIDDEN, 1)
    xsT = lax.dot_general(scw_ref[...], featT, (((1,), (0,)), ((), ())),
                          precision=hi, preferred_element_type=jnp.float32)
    ht = xsT + dxT
    ht_ref[...] = (ht.reshape(8, 8, N, 128).transpose(0, 2, 1, 3)
                   .reshape(8, 1, N, 8, 128))

    pts = pts_ref[...]
    coord = (pts - 0.5) / (1.0 + PADDING) + 0.5
    coord = jnp.clip(coord, 0.0, 1.0 - 1e-6)
    i3 = jnp.clip(jnp.floor(coord * RESO).astype(jnp.int32), 0, RESO - 1)
    idx = (i3[0:1] + RESO * i3[1:2] + (RESO * RESO) * i3[2:3])
    idx2_ref[...] = idx.reshape(1, N, 128)
    idxout_ref[...] = idx.reshape(N, 1, 128)


def _mlp_stage(pts3, feat3, fc0_w, fc0_b, fc1_w, fc1_b, sc_w, off):
    return pl.pallas_call(
        _mlp_body,
        grid=(HALF,),
        in_specs=[
            pl.BlockSpec((3, N, 128), lambda g: (0, 0, g + off)),
            pl.BlockSpec((N, F_DIM, 128), lambda g: (0, 0, g + off)),
            pl.BlockSpec((F_DIM, F_DIM), lambda g: (0, 0)),
            pl.BlockSpec((1, F_DIM), lambda g: (0, 0)),
            pl.BlockSpec((HIDDEN, F_DIM), lambda g: (0, 0)),
            pl.BlockSpec((1, HIDDEN), lambda g: (0, 0)),
            pl.BlockSpec((HIDDEN, F_DIM), lambda g: (0, 0)),
        ],
        out_specs=[
            pl.BlockSpec((8, 1, N, 8, 128), lambda g: (0, g, 0, 0, 0)),
            pl.BlockSpec((1, N, 128), lambda g: (g, 0, 0)),
            pl.BlockSpec((N, 1, 128), lambda g: (0, 0, g)),
        ],
        out_shape=[
            jax.ShapeDtypeStruct((8, HALF, N, 8, 128), jnp.float32),
            jax.ShapeDtypeStruct((HALF, N, 128), jnp.int32),
            jax.ShapeDtypeStruct((N, 1, HALF * 128), jnp.int32),
        ],
        compiler_params=pltpu.CompilerParams(
            dimension_semantics=("arbitrary",)),
    )(pts3, feat3, fc0_w, fc0_b.reshape(1, F_DIM), fc1_w,
      fc1_b.reshape(1, HIDDEN), sc_w)


BANKS = 8
BS = BANKS * S


CH = HALF // GC
NCH = N * CH


def _scatter_max_body(ht_hbm, idx_hbm, out_hbm, acc0, acc1, idxbuf, vbuf,
                      outbuf0, outbuf1, sem):
    cid = lax.axis_index("c")
    sid = lax.axis_index("s")
    wid = sid * 2 + cid
    dh = wid // 4
    dl0 = (wid % 4) * 2
    neg = jnp.full((16,), -jnp.inf, dtype=jnp.float32)
    lane = lax.iota(jnp.int32, 16)
    bankoff = jnp.minimum(lane, 15 - lane) * S

    def _slices(t):
        par = t % 2
        tn = t // CH
        g0 = (t % CH) * GC
        return (par, idx_hbm.at[pl.ds(g0, GC), tn, :],
                ht_hbm.at[dh, pl.ds(g0, GC), tn, pl.ds(dl0, 2), :])

    def issue(t):
        par, isrc, vsrc = _slices(t)
        pltpu.async_copy(isrc, idxbuf.at[par], sem.at[par])
        pltpu.async_copy(vsrc, vbuf.at[par], sem.at[par])

    def wait(t):
        par, isrc, vsrc = _slices(t)
        pltpu.make_async_copy(isrc, idxbuf.at[par], sem.at[par]).wait()
        pltpu.make_async_copy(vsrc, vbuf.at[par], sem.at[par]).wait()
        return par

    issue(0)

    def batch_body(n, _):
        def init_body(j, _):
            for k in range(8):
                sl = pl.ds(j * 128 + k * 16, 16)
                acc0[sl] = neg
                acc1[sl] = neg
            return 0
        lax.fori_loop(0, BS // 128, init_body, 0)

        def chunk_body(cc, _):
            t = n * CH + cc

            @pl.when(t + 1 < NCH)
            def _():
                issue(t + 1)

            par = wait(t)

            def ph_body(ph, _):
                for j in range(8):
                    sl = pl.ds(j * 16, 16)
                    idxv = idxbuf[par, ph, sl]
                    v0 = vbuf[par, ph, 0, sl]
                    v1 = vbuf[par, ph, 1, sl]
                    eq = jnp.flip(idxv) == idxv
                    v0 = jnp.where(eq, jnp.maximum(v0, jnp.flip(v0)), v0)
                    v1 = jnp.where(eq, jnp.maximum(v1, jnp.flip(v1)), v1)
                    addr = idxv + bankoff
                    c0 = plsc.load_gather(acc0, [addr])
                    c1 = plsc.load_gather(acc1, [addr])
                    plsc.store_scatter(acc0, [addr], jnp.maximum(c0, v0))
                    plsc.store_scatter(acc1, [addr], jnp.maximum(c1, v1))
                return 0

            lax.fori_loop(0, GC, ph_body, 0)
            return 0

        lax.fori_loop(0, CH, chunk_body, 0)

        def red_body(sh, _):
            for j in range(8):
                pos = sh * 128 + j * 16
                m0 = acc0[pl.ds(pos, 16)]
                m1 = acc1[pl.ds(pos, 16)]
                for b in range(1, BANKS):
                    m0 = jnp.maximum(m0, acc0[pl.ds(b * S + pos, 16)])
                    m1 = jnp.maximum(m1, acc1[pl.ds(b * S + pos, 16)])
                outbuf0[sh, pl.ds(j * 16, 16)] = m0
                outbuf1[sh, pl.ds(j * 16, 16)] = m1
            return 0
        lax.fori_loop(0, SH, red_body, 0)
        pltpu.sync_copy(outbuf0, out_hbm.at[n, dh, :, dl0, :])
        pltpu.sync_copy(outbuf1, out_hbm.at[n, dh, :, dl0 + 1, :])
        return 0

    lax.fori_loop(0, N, batch_body, 0)


def _scatter_max_stage(ht5, idx3):
    mesh = plsc.VectorSubcoreMesh(core_axis_name="c", subcore_axis_name="s")
    f = pl.kernel(
        _scatter_max_body,
        out_type=jax.ShapeDtypeStruct((N, 8, SH, 8, 128), jnp.float32),
        mesh=mesh,
        scratch_types=[
            pltpu.VMEM((BS,), jnp.float32),
            pltpu.VMEM((BS,), jnp.float32),
            pltpu.VMEM((2, GC, 128), jnp.int32),
            pltpu.VMEM((2, GC, 2, 128), jnp.float32),
            pltpu.VMEM((SH, 128), jnp.float32),
            pltpu.VMEM((SH, 128), jnp.float32),
            pltpu.SemaphoreType.DMA((2,)),
        ],
        compiler_params=pltpu.CompilerParams(needs_layout_passes=False),
    )
    return f(ht5, idx3)


def _conv_body(pa_ref, pb_ref, convw_ref, convb_ref, out_ref):
    blk = jnp.maximum(pa_ref[...], pb_ref[...])
    p = blk.reshape(8, SH, 8, 128).transpose(0, 2, 1, 3).reshape(HIDDEN, S)
    p = jnp.where(p == -jnp.inf, 0.0, p)
    g = lax.dot_general(convw_ref[...], p, (((1,), (0,)), ((), ())),
                        precision=lax.Precision.HIGHEST,
                        preferred_element_type=jnp.float32)
    g = g + convb_ref[...].reshape(C_DIM, 1)
    out_ref[...] = g.reshape(1, C_DIM, S)


def _conv_stage(pooled_a, pooled_b, conv_w, conv_b):
    return pl.pallas_call(
        _conv_body,
        grid=(N,),
        in_specs=[
            pl.BlockSpec((1, 8, SH, 8, 128), lambda n: (n, 0, 0, 0, 0)),
            pl.BlockSpec((1, 8, SH, 8, 128), lambda n: (n, 0, 0, 0, 0)),
            pl.BlockSpec((C_DIM, HIDDEN), lambda n: (0, 0)),
            pl.BlockSpec((1, C_DIM), lambda n: (0, 0)),
        ],
        out_specs=pl.BlockSpec((1, C_DIM, S), lambda n: (n, 0, 0)),
        out_shape=jax.ShapeDtypeStruct((N, C_DIM, S), jnp.float32),
        compiler_params=pltpu.CompilerParams(
            dimension_semantics=("arbitrary",)),
    )(pooled_a, pooled_b, conv_w, conv_b.reshape(1, C_DIM))


def kernel(normalized_points, feature, fc0_w, fc0_b, fc1_w, fc1_b, sc_w,
           conv_w, conv_b):
    pts3 = jnp.transpose(normalized_points, (2, 0, 1))
    feat3 = jnp.transpose(feature, (0, 2, 1))
    ht_a, idx_a, io_a = _mlp_stage(pts3, feat3, fc0_w, fc0_b, fc1_w,
                                   fc1_b, sc_w, 0)
    pooled_a = _scatter_max_stage(ht_a, idx_a)
    ht_b, idx_b, io_b = _mlp_stage(pts3, feat3, fc0_w, fc0_b, fc1_w,
                                   fc1_b, sc_w, HALF)
    pooled_b = _scatter_max_stage(ht_b, idx_b)
    grid = _conv_stage(pooled_a, pooled_b, conv_w, conv_b)
    index_out = jnp.concatenate([io_a, io_b], axis=2)
    return (index_out, grid.reshape(N, C_DIM, RESO, RESO, RESO))
